# baseline (speedup 1.0000x reference)
import math

import jax
import jax.numpy as jnp
from jax.experimental import pallas as pl
from jax.experimental.pallas import tpu as pltpu

EMB = 512
NCLS = 100
_BB = 128
_LOG2PI = math.log(2.0 * math.pi)


def _fwd_kernel(x_ref, w1_ref, b1_ref, w2_ref, b2_ref, wfc_ref, bfc_ref,
                cls_ref, clsb_ref, mu_ref, lv_ref, lf_ref, out_ref):
    bb = out_ref.shape[0]
    f32 = jnp.float32
    bf16 = jnp.bfloat16

    xr = x_ref[...]
    acc = None
    for ky in range(5):
        xs = xr[ky:ky + 28].reshape(28 * bb, 96)
        d = jnp.dot(xs, w1_ref[ky], preferred_element_type=f32)
        acc = d if acc is None else acc + d
    a = jnp.maximum(acc + b1_ref[...], 0.0)
    a = a.reshape(14, 2 * bb, 1024)
    a = jnp.maximum(a[:, :bb], a[:, bb:])
    p1 = jnp.maximum(a[:, :, :512], a[:, :, 512:])
    p1 = p1.astype(bf16)

    acc2 = None
    for ky in range(5):
        xs = p1[ky:ky + 10].reshape(10 * bb, 512)
        d = jnp.dot(xs, w2_ref[ky], preferred_element_type=f32)
        acc2 = d if acc2 is None else acc2 + d
    b = jnp.maximum(acc2 + b2_ref[...], 0.0)
    b = b.reshape(5, 2 * bb, 768)
    b = jnp.maximum(b[:, :bb], b[:, bb:])
    p2 = jnp.maximum(b[:, :, :384], b[:, :, 384:])
    p2 = p2.astype(bf16)

    z = None
    for y in range(5):
        d = jnp.dot(p2[y], wfc_ref[y], preferred_element_type=f32)
        z = d if z is None else z + d
    e = jnp.maximum(z + bfc_ref[...], 0.0)
    e16 = e.astype(bf16)

    logits = jnp.dot(e16, cls_ref[...], preferred_element_type=f32) + clsb_ref[...]

    lv = lv_ref[...]
    mu_t = mu_ref[...]
    iv = jnp.exp(-lv)
    m1 = (mu_t * iv).astype(bf16)
    iv16 = iv.astype(bf16)
    c2 = jnp.sum(mu_t * mu_t * iv, axis=0, keepdims=True)
    logdet = jnp.sum(lv, axis=0, keepdims=True)
    e2 = (e * e).astype(bf16)
    quad = (jnp.dot(e2, iv16, preferred_element_type=f32)
            - 2.0 * jnp.dot(e16, m1, preferred_element_type=f32) + c2)
    logp = -0.5 * (quad + logdet + EMB * _LOG2PI) + jnp.log(lf_ref[...])
    kmask = jax.lax.broadcasted_iota(jnp.int32, (1, 128), 1) < NCLS
    neg = jnp.float32(-1e30)
    logp = jnp.where(kmask, logp, neg)
    m = jnp.max(logp, axis=1, keepdims=True)
    log_prob = m + jnp.log(jnp.sum(jnp.exp(logp - m), axis=1, keepdims=True))
    evidence = jnp.exp(jnp.clip(log_prob, -30.0, 30.0))

    lg = jnp.where(kmask, logits, neg)
    mm = jnp.max(lg, axis=1, keepdims=True)
    sm = jnp.exp(lg - mm)
    sm = sm / jnp.sum(sm, axis=1, keepdims=True)
    alpha = 1.0 + evidence * sm
    out_ref[...] = alpha[:, :NCLS]


def kernel(x, conv1_w, conv1_b, conv2_w, conv2_b, fc1_w, fc1_b,
           cls_w, cls_b, mu, log_var, label_freq):
    f32 = jnp.float32
    bf16 = jnp.bfloat16
    batch = x.shape[0]

    xt = jnp.transpose(x.astype(bf16), (2, 0, 3, 1)).reshape(32, batch, 96)

    w1t = jnp.transpose(conv1_w, (2, 3, 1, 0))
    d1 = jnp.arange(32)[:, None] - jnp.arange(28)[None, :]
    g1 = w1t[:, jnp.clip(d1, 0, 4)]
    g1 = g1 * ((d1 >= 0) & (d1 < 5))[None, :, :, None, None]
    g1 = g1.transpose(0, 1, 3, 2, 4)
    g1 = g1.reshape(5, 96, 14, 2, 32).transpose(0, 1, 3, 2, 4)
    g1 = g1.reshape(5, 96, 2, 448)
    w1 = jnp.pad(g1, ((0, 0), (0, 0), (0, 0), (0, 64))).reshape(5, 96, 1024)
    w1 = w1.astype(bf16)
    b1 = jnp.tile(jnp.pad(jnp.tile(conv1_b, 14), (0, 64)), 2)[None]

    w2t = jnp.transpose(conv2_w, (2, 3, 1, 0))
    d2 = jnp.arange(14)[:, None] - jnp.arange(10)[None, :]
    g2 = w2t[:, jnp.clip(d2, 0, 4)]
    g2 = g2 * ((d2 >= 0) & (d2 < 5))[None, :, :, None, None]
    g2 = g2.transpose(0, 1, 3, 2, 4)
    g2 = g2.reshape(5, 448, 5, 2, 64).transpose(0, 1, 3, 2, 4)
    g2 = g2.reshape(5, 448, 2, 320)
    w2 = jnp.pad(g2, ((0, 0), (0, 0), (0, 0), (0, 64))).reshape(5, 448, 768)
    w2 = jnp.pad(w2, ((0, 0), (0, 64), (0, 0))).astype(bf16)
    b2 = jnp.tile(jnp.pad(jnp.tile(conv2_b, 5), (0, 64)), 2)[None]

    wfc = fc1_w.reshape(512, 64, 5, 5).transpose(2, 3, 1, 0).reshape(5, 320, 512)
    wfc = jnp.pad(wfc, ((0, 0), (0, 64), (0, 0))).astype(bf16)
    bfc = fc1_b[None]

    clsT = jnp.pad(cls_w.T, ((0, 0), (0, 28))).astype(bf16)
    clsb = jnp.pad(cls_b, (0, 28))[None]
    muT = jnp.pad(mu.T, ((0, 0), (0, 28)))
    lvT = jnp.pad(log_var.T, ((0, 0), (0, 28)))
    lf = jnp.pad(label_freq, (0, 28), constant_values=1.0)[None]

    out = pl.pallas_call(
        _fwd_kernel,
        grid=(batch // _BB,),
        in_specs=[
            pl.BlockSpec((32, _BB, 96), lambda i: (0, i, 0)),
            pl.BlockSpec((5, 96, 1024), lambda i: (0, 0, 0)),
            pl.BlockSpec((1, 1024), lambda i: (0, 0)),
            pl.BlockSpec((5, 512, 768), lambda i: (0, 0, 0)),
            pl.BlockSpec((1, 768), lambda i: (0, 0)),
            pl.BlockSpec((5, 384, 512), lambda i: (0, 0, 0)),
            pl.BlockSpec((1, 512), lambda i: (0, 0)),
            pl.BlockSpec((512, 128), lambda i: (0, 0)),
            pl.BlockSpec((1, 128), lambda i: (0, 0)),
            pl.BlockSpec((512, 128), lambda i: (0, 0)),
            pl.BlockSpec((512, 128), lambda i: (0, 0)),
            pl.BlockSpec((1, 128), lambda i: (0, 0)),
        ],
        out_specs=pl.BlockSpec((_BB, NCLS), lambda i: (i, 0)),
        out_shape=jax.ShapeDtypeStruct((batch, NCLS), f32),
        compiler_params=pltpu.CompilerParams(
            dimension_semantics=("arbitrary",)),
    )(xt, w1, b1, w2, b2, wfc, bfc, clsT, clsb, muT, lvT, lf)
    return out
